# baseline (speedup 1.0000x reference)
import functools

import jax
import jax.numpy as jnp
from jax import lax
from jax.experimental import pallas as pl
from jax.experimental.pallas import tpu as pltpu
from jax.experimental.pallas import tpu_sc as plsc

N = 10000
E = 320000
D = 128
NC = 2
NS = 16
K = 112
CH = 90
E_PAD = NC * NS * CH * K
SG = 6
QG = CH // SG
MQ = QG // 3
RPT = 640
RPT_LAST = N - (NS - 1) * RPT
ZR = 80
MB = 1000


def _mm_body(x_ref, w_ref, o_ref):
    o_ref[...] = jnp.dot(x_ref[...], w_ref[...],
                         preferred_element_type=jnp.float32)


def _support(x, w):
    return pl.pallas_call(
        _mm_body,
        grid=(N // MB,),
        in_specs=[
            pl.BlockSpec((MB, D), lambda i: (i, 0)),
            pl.BlockSpec((D, D), lambda i: (0, 0)),
        ],
        out_specs=pl.BlockSpec((MB, D), lambda i: (i, 0)),
        out_shape=jax.ShapeDtypeStruct((N, D), jnp.float32),
    )(x, w)


def _merge_body(p_ref, b_ref, o_ref):
    o_ref[...] = p_ref[0] + p_ref[1] + b_ref[0]


def _merge(partials, b):
    return pl.pallas_call(
        _merge_body,
        grid=(N // MB,),
        in_specs=[
            pl.BlockSpec((NC, MB, D), lambda i: (0, i, 0)),
            pl.BlockSpec((1, D), lambda i: (0, 0)),
        ],
        out_specs=pl.BlockSpec((MB, D), lambda i: (i, 0)),
        out_shape=jax.ShapeDtypeStruct((N, D), jnp.float32),
    )(partials, b.reshape(1, D))


_mesh = plsc.VectorSubcoreMesh(
    core_axis_name="c", subcore_axis_name="s", num_cores=NC, num_subcores=NS)


@functools.partial(
    pl.kernel,
    out_type=jax.ShapeDtypeStruct((NC, N, D), jnp.float32),
    mesh=_mesh,
    compiler_params=pltpu.CompilerParams(use_tc_tiling_on_sc=False),
    scratch_types=[
        pltpu.VMEM((2, SG, K), jnp.int32),
        pltpu.VMEM((2, SG, K), jnp.int32),
        pltpu.VMEM((2, SG, K), jnp.float32),
        pltpu.VMEM((2, K, D), jnp.float32),
        pltpu.VMEM_SHARED((N, D), jnp.float32),
        [pltpu.SemaphoreType.DMA] * 2,
        [pltpu.SemaphoreType.DMA] * 2,
        [pltpu.SemaphoreType.DMA] * 2,
    ],
)
def _sc_spmm(src_hbm, dst_hbm, val_hbm, sup_hbm, out_hbm,
             src_sb, dst_sb, val_sb, rows_f, acc_sh,
             sem_g, sem_s, sem_i):
    c = lax.axis_index("c")
    s = lax.axis_index("s")

    def start_idx(q, p):
        sl = pl.ds(q * SG, SG)

        def go(pp):
            pltpu.async_copy(src_hbm.at[c, s, sl], src_sb.at[pp], sem_i[pp])
            pltpu.async_copy(dst_hbm.at[c, s, sl], dst_sb.at[pp], sem_i[pp])
            pltpu.async_copy(val_hbm.at[c, s, sl], val_sb.at[pp], sem_i[pp])

        if isinstance(p, int):
            go(p)
        else:
            pl.when(p == 0)(lambda: go(0))
            pl.when(p == 1)(lambda: go(1))

    def wait_idx(p):
        sl = pl.ds(0, SG)

        def go(pp):
            pltpu.make_async_copy(
                src_hbm.at[c, s, sl], src_sb.at[pp], sem_i[pp]).wait()
            pltpu.make_async_copy(
                dst_hbm.at[c, s, sl], dst_sb.at[pp], sem_i[pp]).wait()
            pltpu.make_async_copy(
                val_hbm.at[c, s, sl], val_sb.at[pp], sem_i[pp]).wait()

        if isinstance(p, int):
            go(p)
        else:
            pl.when(p == 0)(lambda: go(0))
            pl.when(p == 1)(lambda: go(1))

    def start_gather(b, p, j):
        def go(bb):
            pltpu.async_copy(sup_hbm.at[src_sb.at[p, j]], rows_f.at[bb],
                             sem_g[bb])

        if isinstance(b, int):
            go(b)
        else:
            pl.when(b == 0)(lambda: go(0))
            pl.when(b == 1)(lambda: go(1))

    def wait_gather(b):
        def go(bb):
            pltpu.make_async_copy(
                sup_hbm.at[src_sb.at[0, 0]], rows_f.at[bb],
                sem_g[bb]).wait()

        if isinstance(b, int):
            go(b)
        else:
            pl.when(b == 0)(lambda: go(0))
            pl.when(b == 1)(lambda: go(1))

    def start_scatter(b, p, j):
        def go(bb):
            pltpu.async_copy(rows_f.at[bb], acc_sh.at[dst_sb.at[p, j]],
                             sem_s[bb], add=True)

        if isinstance(b, int):
            go(b)
        else:
            pl.when(b == 0)(lambda: go(0))
            pl.when(b == 1)(lambda: go(1))

    def wait_scatter(b):
        def go(bb):
            pltpu.make_async_copy(
                rows_f.at[bb], acc_sh.at[dst_sb.at[0, 0]],
                sem_s[bb]).wait()

        if isinstance(b, int):
            go(b)
        else:
            pl.when(b == 0)(lambda: go(0))
            pl.when(b == 1)(lambda: go(1))

    def scale_buf(b, p, j):
        def scale(g, inner):
            vv = val_sb[p, j, pl.ds(g * 16, 16)]
            for el in range(16):
                vb = jnp.full((16,), vv[el], jnp.float32)
                e = g * 16 + el
                for jj in range(D // 16):
                    sl = pl.ds(16 * jj, 16)
                    rows_f[b, e, sl] = rows_f[b, e, sl] * vb
            return inner

        lax.fori_loop(0, K // 16, scale, 0)

    start_idx(0, 0)
    zvec = jnp.zeros((16,), jnp.float32)

    def zfill(i, carry):
        for j in range(D // 16):
            rows_f[1, i, pl.ds(16 * j, 16)] = zvec
        return carry

    lax.fori_loop(0, ZR, zfill, 0)
    wait_idx(0)
    start_gather(0, 0, 0)
    zsrc = rows_f.at[1, pl.ds(0, ZR)]

    @pl.when(s < NS - 1)
    def _():
        for r in range(RPT // ZR):
            pltpu.sync_copy(zsrc, acc_sh.at[pl.ds(s * RPT + r * ZR, ZR)])

    @pl.when(s == NS - 1)
    def _():
        for r in range(RPT_LAST // ZR):
            pltpu.sync_copy(
                zsrc, acc_sh.at[pl.ds((NS - 1) * RPT + r * ZR, ZR)])

    plsc.subcore_barrier()

    def chunk(i, carry):
        q = i // SG
        j = i - q * SG
        b = lax.rem(i, 2)
        b2 = 1 - b
        p = lax.rem(q, 2)

        @pl.when(jnp.logical_and(j == 2, q < QG - 1))
        def _():
            start_idx(q + 1, 1 - p)

        wait_gather(b)

        @pl.when(i >= 1)
        def _():
            wait_scatter(b2)

        @pl.when(jnp.logical_and(j == 4, q < QG - 1))
        def _():
            wait_idx(1 - p)

        @pl.when(i + 1 < CH)
        def _():
            i2 = i + 1
            q2 = i2 // SG
            start_gather(b2, lax.rem(q2, 2), i2 - q2 * SG)

        scale_buf(b, p, j)
        start_scatter(b, p, j)
        return carry

    lax.fori_loop(0, CH, chunk, 0)
    wait_scatter((CH - 1) % 2)
    plsc.subcore_barrier()

    @pl.when(s < NS - 1)
    def _():
        pltpu.sync_copy(acc_sh.at[pl.ds(s * RPT, RPT)],
                        out_hbm.at[c, pl.ds(s * RPT, RPT)])

    @pl.when(s == NS - 1)
    def _():
        pltpu.sync_copy(acc_sh.at[pl.ds((NS - 1) * RPT, RPT_LAST)],
                        out_hbm.at[c, pl.ds((NS - 1) * RPT, RPT_LAST)])


def kernel(edge_index, edge_vals, in_feature, W, b):
    edge_index = edge_index.astype(jnp.int32)
    pad = E_PAD - E
    idx_pad = jnp.arange(pad, dtype=jnp.int32) % N
    src = jnp.concatenate([edge_index[1], idx_pad]).reshape(NC, NS, CH, K)
    dst = jnp.concatenate([edge_index[0], idx_pad]).reshape(NC, NS, CH, K)
    val = jnp.pad(edge_vals, (0, pad)).reshape(NC, NS, CH, K)
    sup = _support(in_feature, W)
    partials = _sc_spmm(src, dst, val, sup)
    return _merge(partials, b)


# speedup vs baseline: 2.5710x; 2.5710x over previous
import functools

import jax
import jax.numpy as jnp
from jax import lax
from jax.experimental import pallas as pl
from jax.experimental.pallas import tpu as pltpu
from jax.experimental.pallas import tpu_sc as plsc

N = 10000
E = 320000
D = 128
NC = 2
NS = 16
K = 128
CH = 80
E_PAD = NC * NS * CH * K
SG = 8
QG = CH // SG
RPT = 640
RPT_LAST = N - (NS - 1) * RPT
ZR = 80
MB = 1000


def _mm_body(x_ref, w_ref, o_ref):
    o_ref[...] = jnp.dot(x_ref[...], w_ref[...],
                         preferred_element_type=jnp.float32)


def _support(x, w):
    return pl.pallas_call(
        _mm_body,
        grid=(N // MB,),
        in_specs=[
            pl.BlockSpec((MB, D), lambda i: (i, 0)),
            pl.BlockSpec((D, D), lambda i: (0, 0)),
        ],
        out_specs=pl.BlockSpec((MB, D), lambda i: (i, 0)),
        out_shape=jax.ShapeDtypeStruct((N, D), jnp.float32),
    )(x, w)


def _merge_body(p_ref, b_ref, o_ref):
    o_ref[...] = p_ref[0] + p_ref[1] + b_ref[0]


def _merge(partials, b):
    return pl.pallas_call(
        _merge_body,
        grid=(N // MB,),
        in_specs=[
            pl.BlockSpec((NC, MB, D), lambda i: (0, i, 0)),
            pl.BlockSpec((1, D), lambda i: (0, 0)),
        ],
        out_specs=pl.BlockSpec((MB, D), lambda i: (i, 0)),
        out_shape=jax.ShapeDtypeStruct((N, D), jnp.float32),
    )(partials, b.reshape(1, D))


_mesh = plsc.VectorSubcoreMesh(
    core_axis_name="c", subcore_axis_name="s", num_cores=NC, num_subcores=NS)


@functools.partial(
    pl.kernel,
    out_type=jax.ShapeDtypeStruct((NC, N, D), jnp.float32),
    mesh=_mesh,
    scratch_types=[
        pltpu.VMEM((CH, K), jnp.int32),
        pltpu.VMEM((2, SG, K), jnp.int32),
        pltpu.VMEM((2, SG, K), jnp.float32),
        pltpu.VMEM((2, K, D), jnp.float32),
        pltpu.VMEM_SHARED((N, D), jnp.float32),
        [pltpu.SemaphoreType.DMA] * 2,
        [pltpu.SemaphoreType.DMA] * 2,
        [pltpu.SemaphoreType.DMA] * 2,
    ],
)
def _sc_spmm(src_hbm, dst_hbm, val_hbm, sup_hbm, out_hbm,
             src_v, dst_sb, val_sb, rows_v, acc_sh, sem_g, sem_s, sem_i):
    c = lax.axis_index("c")
    s = lax.axis_index("s")

    zvec = jnp.zeros((16,), jnp.float32)

    def zfill(i, carry):
        for j in range(D // 16):
            rows_v[0, i, pl.ds(16 * j, 16)] = zvec
        return carry

    lax.fori_loop(0, ZR, zfill, 0)
    zsrc = rows_v.at[0, pl.ds(0, ZR)]

    @pl.when(s < NS - 1)
    def _():
        for r in range(RPT // ZR):
            pltpu.sync_copy(zsrc, acc_sh.at[pl.ds(s * RPT + r * ZR, ZR)])

    @pl.when(s == NS - 1)
    def _():
        for r in range(RPT_LAST // ZR):
            pltpu.sync_copy(
                zsrc, acc_sh.at[pl.ds((NS - 1) * RPT + r * ZR, ZR)])

    plsc.subcore_barrier()

    pltpu.sync_copy(src_hbm.at[c, s], src_v)

    def start_idx(q, p):
        sl = pl.ds(q * SG, SG)
        pltpu.async_copy(dst_hbm.at[c, s, sl], dst_sb.at[p], sem_i[p])
        pltpu.async_copy(val_hbm.at[c, s, sl], val_sb.at[p], sem_i[p])

    def wait_idx(p):
        pltpu.make_async_copy(
            dst_hbm.at[c, s, pl.ds(0, SG)], dst_sb.at[p], sem_i[p]).wait()
        pltpu.make_async_copy(
            val_hbm.at[c, s, pl.ds(0, SG)], val_sb.at[p], sem_i[p]).wait()

    def start_gather(i, b):
        pltpu.async_copy(sup_hbm.at[src_v.at[i]], rows_v.at[b], sem_g[b])

    def wait_gather(b):
        pltpu.make_async_copy(
            sup_hbm.at[src_v.at[0]], rows_v.at[b], sem_g[b]).wait()

    def start_scatter(b, p, j):
        pltpu.async_copy(rows_v.at[b], acc_sh.at[dst_sb.at[p, j]],
                         sem_s[b], add=True)

    def wait_scatter(b):
        pltpu.make_async_copy(
            rows_v.at[b], acc_sh.at[dst_sb.at[0, 0]], sem_s[b]).wait()

    def scale_buf(b, p, j):
        def scale(g, inner):
            vv = val_sb[p, j, pl.ds(g * 16, 16)]
            for el in range(16):
                vb = jnp.full((16,), vv[el], jnp.float32)
                e = g * 16 + el
                for jj in range(D // 16):
                    sl = pl.ds(16 * jj, 16)
                    rows_v[b, e, sl] = rows_v[b, e, sl] * vb
            return inner

        lax.fori_loop(0, K // 16, scale, 0)

    start_idx(0, 0)
    start_gather(0, 0)

    def group(q2, carry):
        for qq in range(2):
            q = q2 * 2 + qq
            for jj in range(SG // 2):
                for b in range(2):
                    j = jj * 2 + b
                    i = q * SG + j
                    b2 = 1 - b
                    wait_gather(b)
                    if jj == 0 and b == 0:
                        @pl.when(q > 0)
                        def _():
                            wait_scatter(b2)

                        @pl.when(q < QG - 1)
                        def _():
                            start_idx(q + 1, 1 - qq)

                        wait_idx(qq)
                    else:
                        wait_scatter(b2)
                    if qq == 1 and jj == SG // 2 - 1 and b == 1:
                        @pl.when(q2 < QG // 2 - 1)
                        def _():
                            start_gather(i + 1, b2)
                    else:
                        start_gather(i + 1, b2)
                    scale_buf(b, qq, j)
                    start_scatter(b, qq, j)
        return carry

    lax.fori_loop(0, QG // 2, group, 0)
    wait_scatter((CH - 1) % 2)
    plsc.subcore_barrier()

    @pl.when(s < NS - 1)
    def _():
        pltpu.sync_copy(acc_sh.at[pl.ds(s * RPT, RPT)],
                        out_hbm.at[c, pl.ds(s * RPT, RPT)])

    @pl.when(s == NS - 1)
    def _():
        pltpu.sync_copy(acc_sh.at[pl.ds((NS - 1) * RPT, RPT_LAST)],
                        out_hbm.at[c, pl.ds((NS - 1) * RPT, RPT_LAST)])


def kernel(edge_index, edge_vals, in_feature, W, b):
    edge_index = edge_index.astype(jnp.int32)
    pad = E_PAD - E
    idx_pad = jnp.arange(pad, dtype=jnp.int32) % N
    src = jnp.concatenate([edge_index[1], idx_pad]).reshape(NC, NS, CH, K)
    dst = jnp.concatenate([edge_index[0], idx_pad]).reshape(NC, NS, CH, K)
    val = jnp.pad(edge_vals, (0, pad)).reshape(NC, NS, CH, K)
    sup = _support(in_feature, W)
    partials = _sc_spmm(src, dst, val, sup)
    return _merge(partials, b)
